# baseline (speedup 1.0000x reference)
import functools

import jax
import jax.numpy as jnp
from jax import lax
from jax.experimental import pallas as pl
from jax.experimental.pallas import tpu as pltpu
from jax.experimental.pallas import tpu_sc as plsc

_NC, _NS, _L = 2, 16, 16
_NW = _NC * _NS


def _sc_maxrel(xTw_pad, eidx_pad, npad, cw):
    chunk = npad // _NW
    k2 = eidx_pad.shape[1]
    kk = k2 // 2
    grp = cw // _L
    outb = 16
    nblk = chunk // outb
    mesh = plsc.VectorSubcoreMesh(core_axis_name="c", subcore_axis_name="s")

    @functools.partial(
        pl.kernel,
        out_type=jax.ShapeDtypeStruct((npad, cw), jnp.int32),
        mesh=mesh,
        scratch_types=[
            pltpu.VMEM((2, outb, k2), jnp.int32),
            pltpu.VMEM((2, k2, 2 * cw), jnp.int32),
            pltpu.VMEM((2, outb, cw), jnp.int32),
            pltpu.VMEM_SHARED((npad, 2 * cw), jnp.int32),
            pltpu.SemaphoreType.DMA,
            pltpu.SemaphoreType.DMA,
            pltpu.SemaphoreType.DMA,
            pltpu.SemaphoreType.DMA,
            pltpu.SemaphoreType.DMA,
            pltpu.SemaphoreType.DMA,
        ],
    )
    def body(xT_hbm, eidx_hbm, out_hbm, idx_v, rows_v, out_v, tbl_s,
             gsem0, gsem1, osem0, osem1, isem0, isem1):
        sid = lax.axis_index("s")
        wid = sid * _NC + lax.axis_index("c")
        base = wid * chunk
        stg = npad // _NS
        pltpu.sync_copy(
            xT_hbm.at[pl.ds(sid * stg, stg)], tbl_s.at[pl.ds(sid * stg, stg)]
        )
        plsc.subcore_barrier()
        gsems = (gsem0, gsem1)
        osems = (osem0, osem1)
        isems = (isem0, isem1)

        def istart(bk, islot):
            pltpu.async_copy(
                eidx_hbm.at[pl.ds(base + bk * outb, outb)],
                idx_v.at[islot],
                isems[islot],
            )

        def iwait(islot):
            pltpu.make_async_copy(
                eidx_hbm.at[pl.ds(base, outb)], idx_v.at[islot], isems[islot]
            ).wait()

        def start(loc, gslot, islot):
            pltpu.async_copy(
                tbl_s.at[idx_v.at[islot, loc]], rows_v.at[gslot], gsems[gslot]
            )

        def wait(gslot):
            pltpu.make_async_copy(
                tbl_s.at[idx_v.at[0, 0]], rows_v.at[gslot], gsems[gslot]
            ).wait()

        def compute(gslot, oslot, loc):
            def halves(k, g):
                w = rows_v[gslot, k, pl.ds(_L * g, _L)]
                hi = lax.bitcast_convert_type(w, jnp.float32)
                lo = lax.bitcast_convert_type(w << 16, jnp.float32)
                return hi, lo

            def d(k, g):
                jhi, jlo = halves(k, g)
                ihi, ilo = halves(kk + k, g)
                return jhi - ihi, jlo - ilo

            acc = [d(0, g) for g in range(grp)]
            for k in range(1, kk):
                for g in range(grp):
                    dhi, dlo = d(k, g)
                    acc[g] = (
                        jnp.maximum(acc[g][0], dhi),
                        jnp.maximum(acc[g][1], dlo),
                    )
            for g in range(grp):
                hi_bits = lax.bitcast_convert_type(
                    acc[g][0], jnp.int32
                ) & jnp.int32(-65536)
                lo_bits = lax.shift_right_logical(
                    lax.bitcast_convert_type(acc[g][1], jnp.int32), 16
                )
                out_v[oslot, loc, pl.ds(_L * g, _L)] = hi_bits | lo_bits

        def oscatter(bk, oslot):
            pltpu.async_copy(
                out_v.at[oslot],
                out_hbm.at[pl.ds(base + bk * outb, outb)],
                osems[oslot],
            )

        def owait(oslot):
            pltpu.make_async_copy(
                out_v.at[oslot], out_hbm.at[pl.ds(0, outb)], osems[oslot]
            ).wait()

        istart(0, 0)

        def block(bk, oslot):
            @pl.when(bk + 1 < nblk)
            def _():
                istart(bk + 1, 1 - oslot)

            iwait(oslot)

            @pl.when(bk >= 2)
            def _():
                owait(oslot)

            start(0, 0, oslot)

            def pair_body(i, carry):
                l0 = 2 * i
                start(l0 + 1, 1, oslot)
                wait(0)
                compute(0, oslot, l0)

                @pl.when(l0 + 2 < outb)
                def _():
                    start(l0 + 2, 0, oslot)

                wait(1)
                compute(1, oslot, l0 + 1)
                return carry

            lax.fori_loop(0, outb // 2, pair_body, 0)
            oscatter(bk, oslot)

        def outer(m, carry):
            block(2 * m, 0)
            block(2 * m + 1, 1)
            return carry

        lax.fori_loop(0, nblk // 2, outer, 0)
        owait(0)
        owait(1)

    return body(xTw_pad, eidx_pad)


def _tc_conv(xT_pad, y_pad, waT, wbT, brow, n):
    c = xT_pad.shape[1]
    out_c = waT.shape[1]
    bn = 1024

    def body(xT_ref, y_ref, waT_ref, wbT_ref, b_ref, o_ref):
        acc = jnp.dot(xT_ref[...], waT_ref[...], preferred_element_type=jnp.float32)
        acc = acc + jnp.dot(
            y_ref[...].astype(jnp.float32), wbT_ref[...],
            preferred_element_type=jnp.float32,
        )
        o_ref[...] = jnp.maximum(acc + b_ref[...], 0.0)

    return pl.pallas_call(
        body,
        grid=(pl.cdiv(n, bn),),
        in_specs=[
            pl.BlockSpec((bn, c), lambda i: (i, 0)),
            pl.BlockSpec((bn, c), lambda i: (i, 0)),
            pl.BlockSpec((c, out_c), lambda i: (0, 0)),
            pl.BlockSpec((c, out_c), lambda i: (0, 0)),
            pl.BlockSpec((1, out_c), lambda i: (0, 0)),
        ],
        out_specs=pl.BlockSpec((bn, out_c), lambda i: (i, 0)),
        out_shape=jax.ShapeDtypeStruct((n, out_c), jnp.float32),
    )(xT_pad, y_pad, waT, wbT, brow)


def kernel(x, edge_index, W, b):
    _, c, n = x.shape
    npad = -(-n // (8 * _NW)) * (8 * _NW)
    xT = jnp.pad(x[0].T, ((0, npad - n), (0, 0)))
    e0 = edge_index[0, 0]
    e1 = edge_index[1, 0]
    eidx = jnp.pad(
        jnp.concatenate([e0, e1], axis=1), ((0, npad - n), (0, 0))
    )
    xTw = jax.lax.bitcast_convert_type(
        xT.astype(jnp.bfloat16).reshape(npad, c // 2, 2), jnp.int32
    )
    xTw = jnp.concatenate([xTw, xTw], axis=1)
    y_words = _sc_maxrel(xTw, eidx, npad, c // 2)
    y_pad = jax.lax.bitcast_convert_type(y_words, jnp.bfloat16).reshape(npad, c)

    w2 = W[:, :, 0]
    waT = w2[:, :c].T
    wbT = w2[:, c:].T
    outT = _tc_conv(xT, y_pad, waT, wbT, b[None, :], n)
    return jnp.transpose(outT)[None]


# speedup vs baseline: 1.2314x; 1.2314x over previous
import functools

import jax
import jax.numpy as jnp
from jax import lax
from jax.experimental import pallas as pl
from jax.experimental.pallas import tpu as pltpu
from jax.experimental.pallas import tpu_sc as plsc

_NC, _NS, _L = 2, 16, 16
_NW = _NC * _NS


def _sc_maxrel(xTw_pad, eidx_pad, npad, cw):
    chunk = npad // _NW
    k2 = eidx_pad.shape[1]
    kk = k2 // 2
    grp = cw // _L
    outb = 16
    nblk = chunk // outb
    mesh = plsc.VectorSubcoreMesh(core_axis_name="c", subcore_axis_name="s")

    @functools.partial(
        pl.kernel,
        out_type=jax.ShapeDtypeStruct((npad, cw), jnp.int32),
        mesh=mesh,
        compiler_params=pltpu.CompilerParams(use_tc_tiling_on_sc=False),
        scratch_types=[
            pltpu.VMEM((2, outb, k2), jnp.int32),
            pltpu.VMEM((2, k2, cw), jnp.int32),
            pltpu.VMEM((2, outb, cw), jnp.int32),
            pltpu.VMEM_SHARED((npad, cw), jnp.int32),
            pltpu.SemaphoreType.DMA,
            pltpu.SemaphoreType.DMA,
            pltpu.SemaphoreType.DMA,
            pltpu.SemaphoreType.DMA,
            pltpu.SemaphoreType.DMA,
            pltpu.SemaphoreType.DMA,
        ],
    )
    def body(xT_hbm, eidx_hbm, out_hbm, idx_v, rows_v, out_v, tbl_s,
             gsem0, gsem1, osem0, osem1, isem0, isem1):
        sid = lax.axis_index("s")
        wid = sid * _NC + lax.axis_index("c")
        base = wid * chunk
        stg = npad // _NS
        pltpu.sync_copy(
            xT_hbm.at[pl.ds(sid * stg, stg)], tbl_s.at[pl.ds(sid * stg, stg)]
        )
        plsc.subcore_barrier()
        gsems = (gsem0, gsem1)
        osems = (osem0, osem1)
        isems = (isem0, isem1)

        def istart(bk, islot):
            pltpu.async_copy(
                eidx_hbm.at[pl.ds(base + bk * outb, outb)],
                idx_v.at[islot],
                isems[islot],
            )

        def iwait(islot):
            pltpu.make_async_copy(
                eidx_hbm.at[pl.ds(base, outb)], idx_v.at[islot], isems[islot]
            ).wait()

        def start(loc, gslot, islot):
            pltpu.async_copy(
                tbl_s.at[idx_v.at[islot, loc]], rows_v.at[gslot], gsems[gslot]
            )

        def wait(gslot):
            pltpu.make_async_copy(
                tbl_s.at[idx_v.at[0, 0]], rows_v.at[gslot], gsems[gslot]
            ).wait()

        def compute(gslot, oslot, loc):
            def halves(k, g):
                w = rows_v[gslot, k, pl.ds(_L * g, _L)]
                hi = lax.bitcast_convert_type(w, jnp.float32)
                lo = lax.bitcast_convert_type(w << 16, jnp.float32)
                return hi, lo

            def d(k, g):
                jhi, jlo = halves(k, g)
                ihi, ilo = halves(kk + k, g)
                return jhi - ihi, jlo - ilo

            acc = [d(0, g) for g in range(grp)]
            for k in range(1, kk):
                for g in range(grp):
                    dhi, dlo = d(k, g)
                    acc[g] = (
                        jnp.maximum(acc[g][0], dhi),
                        jnp.maximum(acc[g][1], dlo),
                    )
            for g in range(grp):
                hi_bits = lax.bitcast_convert_type(
                    acc[g][0], jnp.int32
                ) & jnp.int32(-65536)
                lo_bits = lax.shift_right_logical(
                    lax.bitcast_convert_type(acc[g][1], jnp.int32), 16
                )
                out_v[oslot, loc, pl.ds(_L * g, _L)] = hi_bits | lo_bits

        def oscatter(bk, oslot):
            pltpu.async_copy(
                out_v.at[oslot],
                out_hbm.at[pl.ds(base + bk * outb, outb)],
                osems[oslot],
            )

        def owait(oslot):
            pltpu.make_async_copy(
                out_v.at[oslot], out_hbm.at[pl.ds(0, outb)], osems[oslot]
            ).wait()

        istart(0, 0)

        def block(bk, oslot):
            @pl.when(bk + 1 < nblk)
            def _():
                istart(bk + 1, 1 - oslot)

            iwait(oslot)

            @pl.when(bk >= 2)
            def _():
                owait(oslot)

            start(0, 0, oslot)

            def pair_body(i, carry):
                l0 = 2 * i
                start(l0 + 1, 1, oslot)
                wait(0)
                compute(0, oslot, l0)

                @pl.when(l0 + 2 < outb)
                def _():
                    start(l0 + 2, 0, oslot)

                wait(1)
                compute(1, oslot, l0 + 1)
                return carry

            lax.fori_loop(0, outb // 2, pair_body, 0)
            oscatter(bk, oslot)

        def outer(m, carry):
            block(2 * m, 0)
            block(2 * m + 1, 1)
            return carry

        lax.fori_loop(0, nblk // 2, outer, 0)
        owait(0)
        owait(1)

    return body(xTw_pad, eidx_pad)


def _tc_conv(xT_pad, y_pad, waT, wbT, brow, n):
    c = xT_pad.shape[1]
    out_c = waT.shape[1]
    bn = 1024

    def body(xT_ref, y_ref, waT_ref, wbT_ref, b_ref, o_ref):
        acc = jnp.dot(xT_ref[...], waT_ref[...], preferred_element_type=jnp.float32)
        acc = acc + jnp.dot(
            y_ref[...].astype(jnp.float32), wbT_ref[...],
            preferred_element_type=jnp.float32,
        )
        o_ref[...] = jnp.maximum(acc + b_ref[...], 0.0)

    return pl.pallas_call(
        body,
        grid=(pl.cdiv(n, bn),),
        in_specs=[
            pl.BlockSpec((bn, c), lambda i: (i, 0)),
            pl.BlockSpec((bn, c), lambda i: (i, 0)),
            pl.BlockSpec((c, out_c), lambda i: (0, 0)),
            pl.BlockSpec((c, out_c), lambda i: (0, 0)),
            pl.BlockSpec((1, out_c), lambda i: (0, 0)),
        ],
        out_specs=pl.BlockSpec((bn, out_c), lambda i: (i, 0)),
        out_shape=jax.ShapeDtypeStruct((n, out_c), jnp.float32),
    )(xT_pad, y_pad, waT, wbT, brow)


def kernel(x, edge_index, W, b):
    _, c, n = x.shape
    npad = -(-n // (8 * _NW)) * (8 * _NW)
    xT = jnp.pad(x[0].T, ((0, npad - n), (0, 0)))
    e0 = edge_index[0, 0]
    e1 = edge_index[1, 0]
    eidx = jnp.pad(
        jnp.concatenate([e0, e1], axis=1), ((0, npad - n), (0, 0))
    )
    xTw = jax.lax.bitcast_convert_type(
        xT.astype(jnp.bfloat16).reshape(npad, c // 2, 2), jnp.int32
    )
    y_words = _sc_maxrel(xTw, eidx, npad, c // 2)
    y_pad = jax.lax.bitcast_convert_type(y_words, jnp.bfloat16).reshape(npad, c)

    w2 = W[:, :, 0]
    waT = w2[:, :c].T
    wbT = w2[:, c:].T
    outT = _tc_conv(xT, y_pad, waT, wbT, b[None, :], n)
    return jnp.transpose(outT)[None]


# speedup vs baseline: 1.5917x; 1.2926x over previous
import functools

import jax
import jax.numpy as jnp
from jax import lax
from jax.experimental import pallas as pl
from jax.experimental.pallas import tpu as pltpu
from jax.experimental.pallas import tpu_sc as plsc

_NC, _NS, _L = 2, 16, 16
_NW = _NC * _NS


def _sc_maxrel(xTw_pad, eidx_pad, npad, cw):
    chunk = npad // _NW
    k2 = eidx_pad.shape[1]
    kk = k2 // 2
    grp = cw // _L
    outb = 16
    nblk = chunk // outb
    mesh = plsc.VectorSubcoreMesh(core_axis_name="c", subcore_axis_name="s")

    @functools.partial(
        pl.kernel,
        out_type=jax.ShapeDtypeStruct((npad, cw), jnp.int32),
        mesh=mesh,
        compiler_params=pltpu.CompilerParams(use_tc_tiling_on_sc=False),
        scratch_types=[
            pltpu.VMEM((2, outb, k2), jnp.int32),
            pltpu.VMEM((2, k2, cw), jnp.int32),
            pltpu.VMEM((2, outb, cw), jnp.int32),
            pltpu.VMEM_SHARED((npad, cw), jnp.int32),
            pltpu.SemaphoreType.DMA,
            pltpu.SemaphoreType.DMA,
            pltpu.SemaphoreType.DMA,
            pltpu.SemaphoreType.DMA,
            pltpu.SemaphoreType.DMA,
            pltpu.SemaphoreType.DMA,
        ],
    )
    def body(xT_hbm, eidx_hbm, out_hbm, idx_v, rows_v, out_v, tbl_s,
             gsem0, gsem1, osem0, osem1, isem0, isem1):
        sid = lax.axis_index("s")
        wid = sid * _NC + lax.axis_index("c")
        base = wid * chunk
        stg = npad // _NS
        pltpu.sync_copy(
            xT_hbm.at[pl.ds(sid * stg, stg)], tbl_s.at[pl.ds(sid * stg, stg)]
        )
        plsc.subcore_barrier()
        gsems = (gsem0, gsem1)
        osems = (osem0, osem1)
        isems = (isem0, isem1)

        def istart(bk, islot):
            pltpu.async_copy(
                eidx_hbm.at[pl.ds(base + bk * outb, outb)],
                idx_v.at[islot],
                isems[islot],
            )

        def iwait(islot):
            pltpu.make_async_copy(
                eidx_hbm.at[pl.ds(base, outb)], idx_v.at[islot], isems[islot]
            ).wait()

        def start(loc, gslot, islot):
            pltpu.async_copy(
                tbl_s.at[idx_v.at[islot, loc]], rows_v.at[gslot], gsems[gslot]
            )

        def wait(gslot):
            pltpu.make_async_copy(
                tbl_s.at[idx_v.at[0, 0]], rows_v.at[gslot], gsems[gslot]
            ).wait()

        def compute(gslot, oslot, loc):
            def halves(k, g):
                w = rows_v[gslot, k, pl.ds(_L * g, _L)]
                hi = lax.bitcast_convert_type(w, jnp.float32)
                lo = lax.bitcast_convert_type(w << 16, jnp.float32)
                return hi, lo

            def d(k, g):
                jhi, jlo = halves(k, g)
                ihi, ilo = halves(kk + k, g)
                return jhi - ihi, jlo - ilo

            acc = [d(0, g) for g in range(grp)]
            for k in range(1, kk):
                for g in range(grp):
                    dhi, dlo = d(k, g)
                    acc[g] = (
                        jnp.maximum(acc[g][0], dhi),
                        jnp.maximum(acc[g][1], dlo),
                    )
            for g in range(grp):
                hi_bits = lax.bitcast_convert_type(
                    acc[g][0], jnp.int32
                ) & jnp.int32(-65536)
                lo_bits = lax.shift_right_logical(
                    lax.bitcast_convert_type(acc[g][1], jnp.int32), 16
                )
                out_v[oslot, loc, pl.ds(_L * g, _L)] = hi_bits | lo_bits

        def oscatter(bk, oslot):
            pltpu.async_copy(
                out_v.at[oslot],
                out_hbm.at[pl.ds(base + bk * outb, outb)],
                osems[oslot],
            )

        def owait(oslot):
            pltpu.make_async_copy(
                out_v.at[oslot], out_hbm.at[pl.ds(0, outb)], osems[oslot]
            ).wait()

        istart(0, 0)

        def block(bk, oslot):
            @pl.when(bk + 1 < nblk)
            def _():
                istart(bk + 1, 1 - oslot)

            iwait(oslot)

            @pl.when(bk >= 2)
            def _():
                owait(oslot)

            start(0, 0, oslot)

            def pair_body(i, carry):
                l0 = 2 * i
                start(l0 + 1, 1, oslot)
                wait(0)
                compute(0, oslot, l0)

                @pl.when(l0 + 2 < outb)
                def _():
                    start(l0 + 2, 0, oslot)

                wait(1)
                compute(1, oslot, l0 + 1)
                return carry

            lax.fori_loop(0, outb // 2, pair_body, 0)
            oscatter(bk, oslot)

        def outer(m, carry):
            block(2 * m, 0)
            block(2 * m + 1, 1)
            return carry

        lax.fori_loop(0, nblk // 2, outer, 0)
        owait(0)
        owait(1)

    return body(xTw_pad, eidx_pad)


def _tc_pack(xT, npad):
    n, c = xT.shape
    cw = c // 2
    bn = 1024

    def rnd(v):
        i = lax.bitcast_convert_type(v, jnp.int32)
        odd = lax.shift_right_logical(i, 16) & jnp.int32(1)
        return (i + jnp.int32(0x7FFF) + odd) & jnp.int32(-65536)

    def body(x_ref, o_ref):
        lo = rnd(x_ref[:, :cw])
        hi = rnd(x_ref[:, cw:])
        o_ref[...] = hi | lax.shift_right_logical(lo, 16)

    return pl.pallas_call(
        body,
        grid=(npad // bn,),
        in_specs=[pl.BlockSpec((bn, c), lambda i: (i, 0))],
        out_specs=pl.BlockSpec((bn, cw), lambda i: (i, 0)),
        out_shape=jax.ShapeDtypeStruct((npad, cw), jnp.int32),
    )(xT)


def _tc_conv(xT, y_words, waT, wbT, brow, n):
    c = xT.shape[1]
    out_c = waT.shape[1]
    bn = 1024

    def body(xT_ref, y_ref, waT_ref, wbT_ref, b_ref, o_ref):
        w = y_ref[...]
        ylo = lax.bitcast_convert_type(w << 16, jnp.float32)
        yhi = lax.bitcast_convert_type(w & jnp.int32(-65536), jnp.float32)
        y = jnp.concatenate([ylo, yhi], axis=1)
        acc = jnp.dot(xT_ref[...], waT_ref[...], preferred_element_type=jnp.float32)
        acc = acc + jnp.dot(y, wbT_ref[...], preferred_element_type=jnp.float32)
        o_ref[...] = jnp.maximum(acc + b_ref[...], 0.0)

    return pl.pallas_call(
        body,
        grid=(pl.cdiv(n, bn),),
        in_specs=[
            pl.BlockSpec((bn, c), lambda i: (i, 0)),
            pl.BlockSpec((bn, c // 2), lambda i: (i, 0)),
            pl.BlockSpec((c, out_c), lambda i: (0, 0)),
            pl.BlockSpec((c, out_c), lambda i: (0, 0)),
            pl.BlockSpec((1, out_c), lambda i: (0, 0)),
        ],
        out_specs=pl.BlockSpec((bn, out_c), lambda i: (i, 0)),
        out_shape=jax.ShapeDtypeStruct((n, out_c), jnp.float32),
    )(xT, y_words, waT, wbT, brow)


def kernel(x, edge_index, W, b):
    _, c, n = x.shape
    npad = -(-n // (8 * _NW)) * (8 * _NW)
    xT = x[0].T
    e0 = edge_index[0, 0]
    e1 = edge_index[1, 0]
    eidx = jnp.pad(
        jnp.concatenate([e0, e1], axis=1), ((0, npad - n), (0, 0))
    )
    xTw = _tc_pack(xT, npad)
    y_words = _sc_maxrel(xTw, eidx, npad, c // 2)

    w2 = W[:, :, 0]
    waT = w2[:, :c].T
    wbT = w2[:, c:].T
    outT = _tc_conv(xT, y_words, waT, wbT, b[None, :], n)
    return jnp.transpose(outT)[None]
